# baseline (speedup 1.0000x reference)
import functools

import jax
import jax.numpy as jnp
from jax import lax
from jax.experimental import pallas as pl
from jax.experimental.pallas import tpu as pltpu
from jax.experimental.pallas import tpu_sc as plsc

KNbr = 20
LANES = 16


def _knn_body(xr_ref, xf_ref, idx_ref, *, n, tr, k):
    b = pl.program_id(0)
    xr = xr_ref[0]
    xf = xf_ref[0]
    inner = jnp.dot(xr.astype(jnp.bfloat16), xf.T.astype(jnp.bfloat16),
                    preferred_element_type=jnp.float32)
    xxr = jnp.sum(xr * xr, axis=1)[:, None]
    xxf = jnp.sum(xf * xf, axis=1)[None, :]
    negd = -(xxr - 2.0 * inner + xxf)
    col = lax.broadcasted_iota(jnp.int32, (tr, n), 1)
    krow = lax.broadcasted_iota(jnp.int32, (k, tr), 0)
    idx_acc = jnp.zeros((k, tr), jnp.int32)
    for t in range(k):
        amin = jnp.argmax(negd, axis=1).astype(jnp.int32)
        idx_acc = jnp.where(krow == t, amin[None, :], idx_acc)
        negd = jnp.where(col == amin[:, None], -jnp.inf, negd)
    idx_ref[...] = idx_acc + b * n


def _knn(x, k):
    bsz, n, c = x.shape
    tr = 1024
    nt = n // tr
    grid = (bsz, nt)
    return pl.pallas_call(
        functools.partial(_knn_body, n=n, tr=tr, k=k),
        grid=grid,
        in_specs=[
            pl.BlockSpec((1, tr, c), lambda b, r: (b, r, 0)),
            pl.BlockSpec((1, n, c), lambda b, r: (b, 0, 0)),
        ],
        out_specs=pl.BlockSpec((k, tr), lambda b, r: (0, b * nt + r)),
        out_shape=jax.ShapeDtypeStruct((k, bsz * n), jnp.int32),
    )(x, x)


def _sc_gather_rows(h, idxt_flat, untiled, cout=None, nw=32):
    p, cp = h.shape
    cout = cout or cp
    ppw = p // nw
    ch = 64 if cp <= 64 else 32
    nch = ppw // ch
    info = plsc.get_sparse_core_info()
    nc = info.num_cores
    mesh = plsc.VectorSubcoreMesh(core_axis_name="c", subcore_axis_name="s")
    cparams = (pltpu.CompilerParams(use_tc_tiling_on_sc=False)
               if untiled else None)

    @functools.partial(
        pl.kernel,
        out_type=jax.ShapeDtypeStruct((KNbr, p, cout), jnp.float32),
        mesh=mesh,
        compiler_params=cparams,
        scratch_types=[
            pltpu.VMEM((KNbr * ppw,), jnp.int32),
            pltpu.VMEM((KNbr * ch, cp), jnp.float32),
            pltpu.SemaphoreType.DMA,
            pltpu.SemaphoreType.DMA,
        ],
    )
    def k_sc(h_hbm, idxt_hbm, g_hbm, idx_v, rows_v, sem_g, sem_p):
        wid = lax.axis_index("s") * nc + lax.axis_index("c")
        pt_base = wid * ppw
        for j in range(KNbr):
            pltpu.sync_copy(idxt_hbm.at[pl.ds(j * p + pt_base, ppw)],
                            idx_v.at[pl.ds(j * ppw, ppw)])

        def chunk_body(ci, _):
            pt0 = ci * ch
            gets = [
                pltpu.async_copy(
                    h_hbm.at[idx_v.at[pl.ds(j * ppw + pt0, ch)]],
                    rows_v.at[pl.ds(j * ch, ch)],
                    sem_g,
                )
                for j in range(KNbr)
            ]
            for cp_ in gets:
                cp_.wait()
            puts = [
                pltpu.async_copy(
                    rows_v.at[pl.ds(j * ch, ch), pl.ds(0, cout)],
                    g_hbm.at[j, pl.ds(pt_base + pt0, ch)],
                    sem_p,
                )
                for j in range(KNbr)
            ]
            for cp_ in puts:
                cp_.wait()
            return 0

        lax.fori_loop(0, nch, chunk_body, 0)

    return k_sc(h, idxt_flat)


def _econv_body(g_ref, h_ref, w_ref, m_ref, sums_ref, acc_ref, *, c, pt):
    i = pl.program_id(0)

    @pl.when(i == 0)
    def _init():
        acc_ref[...] = jnp.zeros_like(acc_ref)

    gb = g_ref[...]
    hc = h_ref[...]
    fn = gb[:, :, :c] - hc[None, :, :]
    fc = jnp.broadcast_to(hc[None, :, :], (KNbr, pt, c))
    feat = jnp.concatenate([fn, fc], axis=2).astype(jnp.bfloat16)
    feat2 = feat.reshape(KNbr * pt, 2 * c)
    y = jnp.dot(feat2, w_ref[...].astype(jnp.bfloat16),
                preferred_element_type=jnp.float32)
    o = y.shape[1]
    y3 = y.reshape(KNbr, pt, o)
    m_ref[...] = jnp.max(y3, axis=0)
    acc_ref[0:1, :] += jnp.sum(y, axis=0, keepdims=True)
    acc_ref[1:2, :] += jnp.sum(y * y, axis=0, keepdims=True)

    @pl.when(i == pl.num_programs(0) - 1)
    def _fin():
        sums_ref[...] = acc_ref[...]


def _econv(g, h, w2c):
    k, p, cp = g.shape
    c = h.shape[1]
    o = w2c.shape[1]
    pt = 256
    return pl.pallas_call(
        functools.partial(_econv_body, c=c, pt=pt),
        grid=(p // pt,),
        in_specs=[
            pl.BlockSpec((k, pt, cp), lambda i: (0, i, 0)),
            pl.BlockSpec((pt, c), lambda i: (i, 0)),
            pl.BlockSpec((2 * c, o), lambda i: (0, 0)),
        ],
        out_specs=[
            pl.BlockSpec((pt, o), lambda i: (i, 0)),
            pl.BlockSpec((8, o), lambda i: (0, 0)),
        ],
        out_shape=[
            jax.ShapeDtypeStruct((p, o), jnp.float32),
            jax.ShapeDtypeStruct((8, o), jnp.float32),
        ],
        scratch_shapes=[pltpu.VMEM((8, o), jnp.float32)],
    )(g, h, w2c)


def _efin_body(m_ref, sums_ref, g_ref, b_ref, *rest, pk, pad, proj):
    if proj:
        wu_ref, wv_ref = rest[0], rest[1]
        rest = rest[2:]
    outs = list(rest)
    x_ref = outs.pop(0)
    mean = sums_ref[0:1, :] / pk
    var = sums_ref[1:2, :] / pk - mean * mean
    denom = jnp.sqrt(var + 1e-5)
    y = ((m_ref[...] - mean) / denom) * g_ref[...] + b_ref[...]
    h = jnp.where(y > 0, y, 0.2 * y)
    x_ref[...] = h
    if pad:
        xp_ref = outs.pop(0)
        pt, o = h.shape
        xp_ref[...] = jnp.concatenate(
            [h, jnp.zeros((pt, pad - o), jnp.float32)], axis=1)
    if proj:
        u_ref = outs.pop(0)
        v_ref = outs.pop(0)
        u_ref[...] = jnp.dot(h, wu_ref[...], preferred_element_type=jnp.float32)
        v_ref[...] = jnp.dot(h, wv_ref[...], preferred_element_type=jnp.float32)


def _efinalize(m, sums, g, b, pad=0, wu=None, wv=None):
    p, o = m.shape
    tile = 2048
    proj = wu is not None
    in_specs = [
        pl.BlockSpec((tile, o), lambda i: (i, 0)),
        pl.BlockSpec((8, o), lambda i: (0, 0)),
        pl.BlockSpec((1, o), lambda i: (0, 0)),
        pl.BlockSpec((1, o), lambda i: (0, 0)),
    ]
    args = [m, sums, g.reshape(1, o), b.reshape(1, o)]
    out_specs = [pl.BlockSpec((tile, o), lambda i: (i, 0))]
    out_shape = [jax.ShapeDtypeStruct((p, o), jnp.float32)]
    if pad:
        out_specs.append(pl.BlockSpec((tile, pad), lambda i: (i, 0)))
        out_shape.append(jax.ShapeDtypeStruct((p, pad), jnp.float32))
    if proj:
        ou, ov = wu.shape[1], wv.shape[1]
        in_specs += [
            pl.BlockSpec((o, ou), lambda i: (0, 0)),
            pl.BlockSpec((o, ov), lambda i: (0, 0)),
        ]
        args += [wu, wv]
        out_specs += [
            pl.BlockSpec((tile, ou), lambda i: (i, 0)),
            pl.BlockSpec((tile, ov), lambda i: (i, 0)),
        ]
        out_shape += [
            jax.ShapeDtypeStruct((p, ou), jnp.float32),
            jax.ShapeDtypeStruct((p, ov), jnp.float32),
        ]
    res = pl.pallas_call(
        functools.partial(_efin_body, pk=float(p * KNbr), pad=pad, proj=proj),
        grid=(p // tile,),
        in_specs=in_specs,
        out_specs=out_specs,
        out_shape=out_shape,
    )(*args)
    return res if isinstance(res, (tuple, list)) else (res,)


def _sc_gather_reduce(u, v, gidx_flat, o, nw=32):
    p, op = u.shape
    ppw = p // nw
    ch = 8
    nbuf = 2
    nfire = (ch * KNbr) // 80
    nch = ppw // ch
    info = plsc.get_sparse_core_info()
    nc = info.num_cores

    mesh = plsc.VectorSubcoreMesh(core_axis_name="c", subcore_axis_name="s")

    @functools.partial(
        pl.kernel,
        out_type=(
            jax.ShapeDtypeStruct((p, o), jnp.float32),
            jax.ShapeDtypeStruct((nw, o), jnp.float32),
            jax.ShapeDtypeStruct((nw, o), jnp.float32),
            jax.ShapeDtypeStruct((nw, o), jnp.float32),
        ),
        mesh=mesh,
        scratch_types=[
            pltpu.VMEM((ppw * KNbr,), jnp.int32),
            pltpu.VMEM((ch * KNbr, op), jnp.float32),
            pltpu.VMEM((ch * KNbr, op), jnp.float32),
            pltpu.VMEM((ch, o), jnp.float32),
            pltpu.VMEM((ch, o), jnp.float32),
            pltpu.VMEM((ch, o), jnp.float32),
            pltpu.VMEM((o,), jnp.float32),
            pltpu.VMEM((o,), jnp.float32),
            pltpu.VMEM((o,), jnp.float32),
            pltpu.SemaphoreType.DMA,
            pltpu.SemaphoreType.DMA,
        ],
    )
    def k_sc(u_hbm, v_hbm, gidx_hbm, m_hbm, a1_hbm, ax_hbm, a2_hbm,
             idx_v, rows0_v, rows1_v, v0_v, v1_v, m_v,
             acc1_v, accx_v, acc2_v, sem0, sem1):
        bufs = (rows0_v, rows1_v)
        vbufs = (v0_v, v1_v)
        sems = (sem0, sem1)
        wid = lax.axis_index("s") * nc + lax.axis_index("c")
        pt_base = wid * ppw
        pltpu.sync_copy(gidx_hbm.at[pl.ds(pt_base * KNbr, ppw * KNbr)], idx_v)

        def zero_body(ob, _):
            z = jnp.zeros((LANES,), jnp.float32)
            acc1_v[pl.ds(ob * LANES, LANES)] = z
            accx_v[pl.ds(ob * LANES, LANES)] = z
            acc2_v[pl.ds(ob * LANES, LANES)] = z
            return 0
        lax.fori_loop(0, o // LANES, zero_body, 0)

        def fire(ci, rows, vrows, sem):
            ibase = ci * ch * KNbr
            for f in range(nfire):
                pltpu.async_copy(
                    u_hbm.at[idx_v.at[pl.ds(ibase + f * 80, 80)]],
                    rows.at[pl.ds(f * 80, 80)],
                    sem,
                )
            pltpu.async_copy(v_hbm.at[pl.ds(pt_base + ci * ch, ch)], vrows,
                             sem)

        def drain(rows, vrows, sem):
            for f in range(nfire):
                pltpu.make_async_copy(
                    u_hbm.at[idx_v.at[pl.ds(f * 80, 80)]],
                    rows.at[pl.ds(f * 80, 80)],
                    sem,
                ).wait()
            pltpu.make_async_copy(v_hbm.at[pl.ds(0, ch)], vrows, sem).wait()

        def reduce_chunk(ci, rows, vrows):
            def pt_body(pt, _):
                row0 = pt * KNbr
                for ob in range(o // LANES):
                    co = ob * LANES
                    x = rows[row0, pl.ds(co, LANES)]
                    m = x
                    s1 = x
                    s2 = x * x
                    for j in range(1, KNbr):
                        xj = rows[row0 + j, pl.ds(co, LANES)]
                        m = jnp.maximum(m, xj)
                        s1 = s1 + xj
                        s2 = s2 + xj * xj
                    vv = vrows[pt, pl.ds(co, LANES)]
                    m_v[pt, pl.ds(co, LANES)] = m
                    acc1_v[pl.ds(co, LANES)] += s1 + KNbr * vv
                    accx_v[pl.ds(co, LANES)] += vv * (2.0 * s1 + KNbr * vv)
                    acc2_v[pl.ds(co, LANES)] += s2
                return 0

            lax.fori_loop(0, ch, pt_body, 0)
            pt0 = pt_base + ci * ch
            pltpu.sync_copy(m_v, m_hbm.at[pl.ds(pt0, ch)])

        for bsl in range(nbuf):
            fire(bsl, bufs[bsl], vbufs[bsl], sems[bsl])

        def outer(ci0, _):
            for bsl in range(nbuf):
                ci = ci0 + bsl
                drain(bufs[bsl], vbufs[bsl], sems[bsl])
                reduce_chunk(ci, bufs[bsl], vbufs[bsl])

                @pl.when(ci + nbuf < nch)
                def _():
                    fire(ci + nbuf, bufs[bsl], vbufs[bsl], sems[bsl])
            return 0

        lax.fori_loop(0, nch // nbuf, lambda i, c: outer(i * nbuf, c), 0)
        pltpu.sync_copy(acc1_v, a1_hbm.at[wid])
        pltpu.sync_copy(accx_v, ax_hbm.at[wid])
        pltpu.sync_copy(acc2_v, a2_hbm.at[wid])

    return k_sc(u, v, gidx_flat)


def _fin4_body(m_ref, v_ref, a1_ref, ax_ref, a2_ref, g_ref, b_ref, x_ref, *,
               pk):
    mean = jnp.sum(a1_ref[...], axis=0, keepdims=True) / pk
    ey2 = (jnp.sum(ax_ref[...], axis=0, keepdims=True) +
           jnp.sum(a2_ref[...], axis=0, keepdims=True)) / pk
    var = ey2 - mean * mean
    denom = jnp.sqrt(var + 1e-5)
    y = ((m_ref[...] + v_ref[...] - mean) / denom) * g_ref[...] + b_ref[...]
    x_ref[...] = jnp.where(y > 0, y, 0.2 * y)


def _finalize4(m, v, a1, ax, a2, g, b):
    p, o = m.shape
    tile = 2048
    return pl.pallas_call(
        functools.partial(_fin4_body, pk=float(p * KNbr)),
        grid=(p // tile,),
        in_specs=[
            pl.BlockSpec((tile, o), lambda i: (i, 0)),
            pl.BlockSpec((tile, o), lambda i: (i, 0)),
            pl.BlockSpec(a1.shape, lambda i: (0, 0)),
            pl.BlockSpec(ax.shape, lambda i: (0, 0)),
            pl.BlockSpec(a2.shape, lambda i: (0, 0)),
            pl.BlockSpec((1, o), lambda i: (0, 0)),
            pl.BlockSpec((1, o), lambda i: (0, 0)),
        ],
        out_specs=pl.BlockSpec((tile, o), lambda i: (i, 0)),
        out_shape=jax.ShapeDtypeStruct((p, o), jnp.float32),
    )(m, v, a1, ax, a2, g.reshape(1, o), b.reshape(1, o))


def _split_w(w, c):
    wn = w[:, :c]
    return wn.T, (w[:, c:] - wn).T


def kernel(x, W1, g1, b1, W2, g2, b2, W3, g3, b3, W4, g4, b4):
    bsz, n, c = x.shape
    p = bsz * n
    x2d = x.reshape(p, c)

    wu4, wv4 = _split_w(W4, W3.shape[0])

    idxt1 = _knn(x, KNbr).reshape(-1)

    h1 = jnp.pad(x2d, ((0, 0), (0, 16 - c)))
    g1e = _sc_gather_rows(h1, idxt1, untiled=True)
    m1, sums1 = _econv(g1e, x2d, W1.T)
    (x1, x1p) = _efinalize(m1, sums1, g1, b1, pad=128)

    g2e = _sc_gather_rows(x1p, idxt1, untiled=False)
    m2, sums2 = _econv(g2e, x1, W2.T)
    (x2, x2p) = _efinalize(m2, sums2, g2, b2, pad=128)

    g3e = _sc_gather_rows(x2p, idxt1, untiled=False)
    m3, sums3 = _econv(g3e, x2, W3.T)
    (x3, u4, v4) = _efinalize(m3, sums3, g3, b3, wu=wu4, wv=wv4)

    idxt4 = _knn(x3.reshape(bsz, n, x3.shape[1]), KNbr)
    gidx4 = idxt4.T.reshape(-1)
    m4, a1, ax, a2 = _sc_gather_reduce(u4, v4, gidx4, W4.shape[0])
    x4 = _finalize4(m4, v4, a1, ax, a2, g4, b4)

    out = jnp.concatenate([x1, x2, x3, x4], axis=-1)
    return out.reshape(bsz, n, out.shape[-1])


# speedup vs baseline: 1.1542x; 1.1542x over previous
import functools

import jax
import jax.numpy as jnp
from jax import lax
from jax.experimental import pallas as pl
from jax.experimental.pallas import tpu as pltpu
from jax.experimental.pallas import tpu_sc as plsc

KNbr = 20
LANES = 16


def _knn_body(xr_ref, xf_ref, idx_ref, *, n, tr, k):
    b = pl.program_id(0)
    xr = xr_ref[0]
    xf = xf_ref[0]
    inner = jnp.dot(xr.astype(jnp.bfloat16), xf.T.astype(jnp.bfloat16),
                    preferred_element_type=jnp.float32)
    xxr = jnp.sum(xr * xr, axis=1)[:, None]
    xxf = jnp.sum(xf * xf, axis=1)[None, :]
    negd = -(xxr - 2.0 * inner + xxf)
    col = lax.broadcasted_iota(jnp.int32, (tr, n), 1)
    krow = lax.broadcasted_iota(jnp.int32, (k, tr), 0)
    idx_acc = jnp.zeros((k, tr), jnp.int32)
    for t in range(k):
        amin = jnp.argmax(negd, axis=1).astype(jnp.int32)
        idx_acc = jnp.where(krow == t, amin[None, :], idx_acc)
        negd = jnp.where(col == amin[:, None], -jnp.inf, negd)
    idx_ref[...] = idx_acc + b * n


def _knn(x, k):
    bsz, n, c = x.shape
    tr = 512
    nt = n // tr
    grid = (bsz, nt)
    return pl.pallas_call(
        functools.partial(_knn_body, n=n, tr=tr, k=k),
        grid=grid,
        in_specs=[
            pl.BlockSpec((1, tr, c), lambda b, r: (b, r, 0)),
            pl.BlockSpec((1, n, c), lambda b, r: (b, 0, 0)),
        ],
        out_specs=pl.BlockSpec((k, tr), lambda b, r: (0, b * nt + r)),
        out_shape=jax.ShapeDtypeStruct((k, bsz * n), jnp.int32),
    )(x, x)


def _sc_gather_rows(h, idxt_flat, untiled, cout=None, nw=32):
    p, cp = h.shape
    cout = cout or cp
    ppw = p // nw
    ch = 64 if cp <= 64 else 32
    nch = ppw // ch
    info = plsc.get_sparse_core_info()
    nc = info.num_cores
    mesh = plsc.VectorSubcoreMesh(core_axis_name="c", subcore_axis_name="s")
    cparams = (pltpu.CompilerParams(use_tc_tiling_on_sc=False)
               if untiled else None)

    @functools.partial(
        pl.kernel,
        out_type=jax.ShapeDtypeStruct((KNbr, p, cout), jnp.float32),
        mesh=mesh,
        compiler_params=cparams,
        scratch_types=[
            pltpu.VMEM((KNbr * ppw,), jnp.int32),
            pltpu.VMEM((KNbr * ch, cp), jnp.float32),
            pltpu.SemaphoreType.DMA,
            pltpu.SemaphoreType.DMA,
        ],
    )
    def k_sc(h_hbm, idxt_hbm, g_hbm, idx_v, rows_v, sem_g, sem_p):
        wid = lax.axis_index("s") * nc + lax.axis_index("c")
        pt_base = wid * ppw
        for j in range(KNbr):
            pltpu.sync_copy(idxt_hbm.at[pl.ds(j * p + pt_base, ppw)],
                            idx_v.at[pl.ds(j * ppw, ppw)])

        def chunk_body(ci, _):
            pt0 = ci * ch
            gets = [
                pltpu.async_copy(
                    h_hbm.at[idx_v.at[pl.ds(j * ppw + pt0, ch)]],
                    rows_v.at[pl.ds(j * ch, ch)],
                    sem_g,
                )
                for j in range(KNbr)
            ]
            for cp_ in gets:
                cp_.wait()
            puts = [
                pltpu.async_copy(
                    rows_v.at[pl.ds(j * ch, ch), pl.ds(0, cout)],
                    g_hbm.at[j, pl.ds(pt_base + pt0, ch)],
                    sem_p,
                )
                for j in range(KNbr)
            ]
            for cp_ in puts:
                cp_.wait()
            return 0

        lax.fori_loop(0, nch, chunk_body, 0)

    return k_sc(h, idxt_flat)


def _econv_body(g_ref, h_ref, w_ref, m_ref, sums_ref, acc_ref, *, c, pt):
    i = pl.program_id(0)

    @pl.when(i == 0)
    def _init():
        acc_ref[...] = jnp.zeros_like(acc_ref)

    gb = g_ref[...]
    hc = h_ref[...]
    fn = gb[:, :, :c] - hc[None, :, :]
    fc = jnp.broadcast_to(hc[None, :, :], (KNbr, pt, c))
    feat = jnp.concatenate([fn, fc], axis=2).astype(jnp.bfloat16)
    feat2 = feat.reshape(KNbr * pt, 2 * c)
    y = jnp.dot(feat2, w_ref[...].astype(jnp.bfloat16),
                preferred_element_type=jnp.float32)
    o = y.shape[1]
    y3 = y.reshape(KNbr, pt, o)
    m_ref[...] = jnp.max(y3, axis=0)
    acc_ref[0:1, :] += jnp.sum(y, axis=0, keepdims=True)
    acc_ref[1:2, :] += jnp.sum(y * y, axis=0, keepdims=True)

    @pl.when(i == pl.num_programs(0) - 1)
    def _fin():
        sums_ref[...] = acc_ref[...]


def _econv(g, h, w2c):
    k, p, cp = g.shape
    c = h.shape[1]
    o = w2c.shape[1]
    pt = 256
    return pl.pallas_call(
        functools.partial(_econv_body, c=c, pt=pt),
        grid=(p // pt,),
        in_specs=[
            pl.BlockSpec((k, pt, cp), lambda i: (0, i, 0)),
            pl.BlockSpec((pt, c), lambda i: (i, 0)),
            pl.BlockSpec((2 * c, o), lambda i: (0, 0)),
        ],
        out_specs=[
            pl.BlockSpec((pt, o), lambda i: (i, 0)),
            pl.BlockSpec((8, o), lambda i: (0, 0)),
        ],
        out_shape=[
            jax.ShapeDtypeStruct((p, o), jnp.float32),
            jax.ShapeDtypeStruct((8, o), jnp.float32),
        ],
        scratch_shapes=[pltpu.VMEM((8, o), jnp.float32)],
    )(g, h, w2c)


def _efin_body(m_ref, sums_ref, g_ref, b_ref, *rest, pk, pad, proj):
    if proj:
        wu_ref, wv_ref = rest[0], rest[1]
        rest = rest[2:]
    outs = list(rest)
    x_ref = outs.pop(0)
    mean = sums_ref[0:1, :] / pk
    var = sums_ref[1:2, :] / pk - mean * mean
    denom = jnp.sqrt(var + 1e-5)
    y = ((m_ref[...] - mean) / denom) * g_ref[...] + b_ref[...]
    h = jnp.where(y > 0, y, 0.2 * y)
    x_ref[...] = h
    if pad:
        xp_ref = outs.pop(0)
        pt, o = h.shape
        xp_ref[...] = jnp.concatenate(
            [h, jnp.zeros((pt, pad - o), jnp.float32)], axis=1)
    if proj:
        u_ref = outs.pop(0)
        v_ref = outs.pop(0)
        u_ref[...] = jnp.dot(h, wu_ref[...], preferred_element_type=jnp.float32)
        v_ref[...] = jnp.dot(h, wv_ref[...], preferred_element_type=jnp.float32)


def _efinalize(m, sums, g, b, pad=0, wu=None, wv=None):
    p, o = m.shape
    tile = 2048
    proj = wu is not None
    in_specs = [
        pl.BlockSpec((tile, o), lambda i: (i, 0)),
        pl.BlockSpec((8, o), lambda i: (0, 0)),
        pl.BlockSpec((1, o), lambda i: (0, 0)),
        pl.BlockSpec((1, o), lambda i: (0, 0)),
    ]
    args = [m, sums, g.reshape(1, o), b.reshape(1, o)]
    out_specs = [pl.BlockSpec((tile, o), lambda i: (i, 0))]
    out_shape = [jax.ShapeDtypeStruct((p, o), jnp.float32)]
    if pad:
        out_specs.append(pl.BlockSpec((tile, pad), lambda i: (i, 0)))
        out_shape.append(jax.ShapeDtypeStruct((p, pad), jnp.float32))
    if proj:
        ou, ov = wu.shape[1], wv.shape[1]
        in_specs += [
            pl.BlockSpec((o, ou), lambda i: (0, 0)),
            pl.BlockSpec((o, ov), lambda i: (0, 0)),
        ]
        args += [wu, wv]
        out_specs += [
            pl.BlockSpec((tile, ou), lambda i: (i, 0)),
            pl.BlockSpec((tile, ov), lambda i: (i, 0)),
        ]
        out_shape += [
            jax.ShapeDtypeStruct((p, ou), jnp.float32),
            jax.ShapeDtypeStruct((p, ov), jnp.float32),
        ]
    res = pl.pallas_call(
        functools.partial(_efin_body, pk=float(p * KNbr), pad=pad, proj=proj),
        grid=(p // tile,),
        in_specs=in_specs,
        out_specs=out_specs,
        out_shape=out_shape,
    )(*args)
    return res if isinstance(res, (tuple, list)) else (res,)


def _sc_gather_reduce(u, v, gidx_flat, o, nw=32):
    p, op = u.shape
    ppw = p // nw
    ch = 8
    nbuf = 2
    nfire = (ch * KNbr) // 80
    nch = ppw // ch
    info = plsc.get_sparse_core_info()
    nc = info.num_cores

    mesh = plsc.VectorSubcoreMesh(core_axis_name="c", subcore_axis_name="s")

    @functools.partial(
        pl.kernel,
        out_type=(
            jax.ShapeDtypeStruct((p, o), jnp.float32),
            jax.ShapeDtypeStruct((nw, o), jnp.float32),
            jax.ShapeDtypeStruct((nw, o), jnp.float32),
            jax.ShapeDtypeStruct((nw, o), jnp.float32),
        ),
        mesh=mesh,
        scratch_types=[
            pltpu.VMEM((ppw * KNbr,), jnp.int32),
            pltpu.VMEM((ch * KNbr, op), jnp.float32),
            pltpu.VMEM((ch * KNbr, op), jnp.float32),
            pltpu.VMEM((ch, o), jnp.float32),
            pltpu.VMEM((ch, o), jnp.float32),
            pltpu.VMEM((ch, o), jnp.float32),
            pltpu.VMEM((o,), jnp.float32),
            pltpu.VMEM((o,), jnp.float32),
            pltpu.VMEM((o,), jnp.float32),
            pltpu.SemaphoreType.DMA,
            pltpu.SemaphoreType.DMA,
        ],
    )
    def k_sc(u_hbm, v_hbm, gidx_hbm, m_hbm, a1_hbm, ax_hbm, a2_hbm,
             idx_v, rows0_v, rows1_v, v0_v, v1_v, m_v,
             acc1_v, accx_v, acc2_v, sem0, sem1):
        bufs = (rows0_v, rows1_v)
        vbufs = (v0_v, v1_v)
        sems = (sem0, sem1)
        wid = lax.axis_index("s") * nc + lax.axis_index("c")
        pt_base = wid * ppw
        pltpu.sync_copy(gidx_hbm.at[pl.ds(pt_base * KNbr, ppw * KNbr)], idx_v)

        def zero_body(ob, _):
            z = jnp.zeros((LANES,), jnp.float32)
            acc1_v[pl.ds(ob * LANES, LANES)] = z
            accx_v[pl.ds(ob * LANES, LANES)] = z
            acc2_v[pl.ds(ob * LANES, LANES)] = z
            return 0
        lax.fori_loop(0, o // LANES, zero_body, 0)

        def fire(ci, rows, vrows, sem):
            ibase = ci * ch * KNbr
            for f in range(nfire):
                pltpu.async_copy(
                    u_hbm.at[idx_v.at[pl.ds(ibase + f * 80, 80)]],
                    rows.at[pl.ds(f * 80, 80)],
                    sem,
                )
            pltpu.async_copy(v_hbm.at[pl.ds(pt_base + ci * ch, ch)], vrows,
                             sem)

        def drain(rows, vrows, sem):
            for f in range(nfire):
                pltpu.make_async_copy(
                    u_hbm.at[idx_v.at[pl.ds(f * 80, 80)]],
                    rows.at[pl.ds(f * 80, 80)],
                    sem,
                ).wait()
            pltpu.make_async_copy(v_hbm.at[pl.ds(0, ch)], vrows, sem).wait()

        def reduce_chunk(ci, rows, vrows):
            def pt_body(pt, _):
                row0 = pt * KNbr
                for ob in range(o // LANES):
                    co = ob * LANES
                    x = rows[row0, pl.ds(co, LANES)]
                    m = x
                    s1 = x
                    s2 = x * x
                    for j in range(1, KNbr):
                        xj = rows[row0 + j, pl.ds(co, LANES)]
                        m = jnp.maximum(m, xj)
                        s1 = s1 + xj
                        s2 = s2 + xj * xj
                    vv = vrows[pt, pl.ds(co, LANES)]
                    m_v[pt, pl.ds(co, LANES)] = m
                    acc1_v[pl.ds(co, LANES)] += s1 + KNbr * vv
                    accx_v[pl.ds(co, LANES)] += vv * (2.0 * s1 + KNbr * vv)
                    acc2_v[pl.ds(co, LANES)] += s2
                return 0

            lax.fori_loop(0, ch, pt_body, 0)
            pt0 = pt_base + ci * ch
            pltpu.sync_copy(m_v, m_hbm.at[pl.ds(pt0, ch)])

        for bsl in range(nbuf):
            fire(bsl, bufs[bsl], vbufs[bsl], sems[bsl])

        def outer(ci0, _):
            for bsl in range(nbuf):
                ci = ci0 + bsl
                drain(bufs[bsl], vbufs[bsl], sems[bsl])
                reduce_chunk(ci, bufs[bsl], vbufs[bsl])

                @pl.when(ci + nbuf < nch)
                def _():
                    fire(ci + nbuf, bufs[bsl], vbufs[bsl], sems[bsl])
            return 0

        lax.fori_loop(0, nch // nbuf, lambda i, c: outer(i * nbuf, c), 0)
        pltpu.sync_copy(acc1_v, a1_hbm.at[wid])
        pltpu.sync_copy(accx_v, ax_hbm.at[wid])
        pltpu.sync_copy(acc2_v, a2_hbm.at[wid])

    return k_sc(u, v, gidx_flat)


def _fin4_body(m_ref, v_ref, a1_ref, ax_ref, a2_ref, g_ref, b_ref, x_ref, *,
               pk):
    mean = jnp.sum(a1_ref[...], axis=0, keepdims=True) / pk
    ey2 = (jnp.sum(ax_ref[...], axis=0, keepdims=True) +
           jnp.sum(a2_ref[...], axis=0, keepdims=True)) / pk
    var = ey2 - mean * mean
    denom = jnp.sqrt(var + 1e-5)
    y = ((m_ref[...] + v_ref[...] - mean) / denom) * g_ref[...] + b_ref[...]
    x_ref[...] = jnp.where(y > 0, y, 0.2 * y)


def _finalize4(m, v, a1, ax, a2, g, b):
    p, o = m.shape
    tile = 2048
    return pl.pallas_call(
        functools.partial(_fin4_body, pk=float(p * KNbr)),
        grid=(p // tile,),
        in_specs=[
            pl.BlockSpec((tile, o), lambda i: (i, 0)),
            pl.BlockSpec((tile, o), lambda i: (i, 0)),
            pl.BlockSpec(a1.shape, lambda i: (0, 0)),
            pl.BlockSpec(ax.shape, lambda i: (0, 0)),
            pl.BlockSpec(a2.shape, lambda i: (0, 0)),
            pl.BlockSpec((1, o), lambda i: (0, 0)),
            pl.BlockSpec((1, o), lambda i: (0, 0)),
        ],
        out_specs=pl.BlockSpec((tile, o), lambda i: (i, 0)),
        out_shape=jax.ShapeDtypeStruct((p, o), jnp.float32),
    )(m, v, a1, ax, a2, g.reshape(1, o), b.reshape(1, o))


def _split_w(w, c):
    wn = w[:, :c]
    return wn.T, (w[:, c:] - wn).T


def kernel(x, W1, g1, b1, W2, g2, b2, W3, g3, b3, W4, g4, b4):
    bsz, n, c = x.shape
    p = bsz * n
    x2d = x.reshape(p, c)

    wu4, wv4 = _split_w(W4, W3.shape[0])

    idxt1 = _knn(x, KNbr).reshape(-1)

    h1 = jnp.pad(x2d, ((0, 0), (0, 16 - c)))
    g1e = _sc_gather_rows(h1, idxt1, untiled=True)
    m1, sums1 = _econv(g1e, x2d, W1.T)
    (x1, x1p) = _efinalize(m1, sums1, g1, b1, pad=128)

    g2e = _sc_gather_rows(x1p, idxt1, untiled=False)
    m2, sums2 = _econv(g2e, x1, W2.T)
    (x2, x2p) = _efinalize(m2, sums2, g2, b2, pad=128)

    g3e = _sc_gather_rows(x2p, idxt1, untiled=False)
    m3, sums3 = _econv(g3e, x2, W3.T)
    (x3, u4, v4) = _efinalize(m3, sums3, g3, b3, wu=wu4, wv=wv4)

    idxt4 = _knn(x3.reshape(bsz, n, x3.shape[1]), KNbr)
    gidx4 = idxt4.T.reshape(-1)
    m4, a1, ax, a2 = _sc_gather_reduce(u4, v4, gidx4, W4.shape[0])
    x4 = _finalize4(m4, v4, a1, ax, a2, g4, b4)

    out = jnp.concatenate([x1, x2, x3, x4], axis=-1)
    return out.reshape(bsz, n, out.shape[-1])
